# baseline (speedup 1.0000x reference)
import functools

import jax
import jax.numpy as jnp
from jax import lax
from jax.experimental import pallas as pl
from jax.experimental.pallas import tpu as pltpu
from jax.experimental.pallas import tpu_sc as plsc

NUM_HIDDENS = 768
LANES = 16
SEG_SLICES = NUM_HIDDENS // LANES
NC, NS = 2, 16
NW = NC * NS
ROWS = 4 * 8192
ROWS_PER_W = ROWS // NW
CHUNK = 64
NPIECE = 2
PIECE = CHUNK // NPIECE
NBUF = 2
NCHUNKS = ROWS_PER_W // CHUNK


def _sc_add(xf, seg2, flag):
    mesh = plsc.VectorSubcoreMesh(core_axis_name="c", subcore_axis_name="s")

    @functools.partial(
        pl.kernel,
        mesh=mesh,
        out_type=jax.ShapeDtypeStruct((ROWS, NUM_HIDDENS), jnp.float32),
        scratch_types=[
            pltpu.VMEM((2, NUM_HIDDENS), jnp.float32),
            pltpu.VMEM((LANES,), jnp.int32),
        ] + [pltpu.VMEM((CHUNK, NUM_HIDDENS), jnp.float32)] * NBUF
          + [pltpu.SemaphoreType.DMA] * (2 * NBUF),
    )
    def k(x_hbm, seg_hbm, flag_hbm, out_hbm, seg_v, flag_v, *ring):
        bufs = ring[:NBUF]
        in_sems = ring[NBUF:2 * NBUF]
        out_sems = ring[2 * NBUF:]
        wid = lax.axis_index("s") * NC + lax.axis_index("c")
        pltpu.sync_copy(seg_hbm, seg_v)
        pltpu.sync_copy(flag_hbm, flag_v)
        f = flag_v[...] != 0
        segs = [
            jnp.where(f, seg_v[0, pl.ds(j * LANES, LANES)],
                      seg_v[1, pl.ds(j * LANES, LANES)])
            for j in range(SEG_SLICES)
        ]
        row0 = wid * ROWS_PER_W

        def in_copy(g, b):
            return pltpu.make_async_copy(
                x_hbm.at[pl.ds(row0 + g * CHUNK, CHUNK)], bufs[b], in_sems[b])

        def out_piece(g, b, h):
            return pltpu.make_async_copy(
                bufs[b].at[pl.ds(h * PIECE, PIECE)],
                out_hbm.at[pl.ds(row0 + g * CHUNK + h * PIECE, PIECE)],
                out_sems[b])

        def compute_piece(b, h):
            buf = bufs[b]

            def row_body(r, c):
                for j in range(SEG_SLICES):
                    sl = pl.ds(j * LANES, LANES)
                    plsc.addupdate(buf.at[r, sl], segs[j])
                return c

            lax.fori_loop(h * PIECE, (h + 1) * PIECE, row_body, 0)

        def process(g, b):
            in_copy(g, b).wait()
            for h in range(NPIECE):
                compute_piece(b, h)
                out_piece(g, b, h).start()

        def wait_out(g, b):
            for h in range(NPIECE):
                out_piece(g, b, h).wait()

        for g in range(NBUF - 1):
            in_copy(g, g).start()
        for g in range(NBUF):
            process(g, g)
            if g >= 1:
                wait_out(g - 1, g - 1)
            in_copy(g + NBUF - 1, (g + NBUF - 1) % NBUF).start()

        def group_body(p, c):
            for b in range(NBUF):
                g = p * NBUF + b
                process(g, b)
                wait_out(g - 1, (b - 1) % NBUF)
                in_copy(g + NBUF - 1, (b - 1) % NBUF).start()
            return c

        lax.fori_loop(1, NCHUNKS // NBUF - 1, group_body, 0)

        for g in range(NCHUNKS - NBUF, NCHUNKS):
            b = g % NBUF
            process(g, b)
            wait_out(g - 1, (b - 1) % NBUF)
            if g + NBUF - 1 < NCHUNKS:
                in_copy(g + NBUF - 1, (b - 1) % NBUF).start()
        wait_out(NCHUNKS - 1, (NCHUNKS - 1) % NBUF)

    return k(xf, seg2, flag)


def kernel(X, seg_emb, first_sentence):
    xf = X.reshape(ROWS, NUM_HIDDENS)
    seg2 = seg_emb.reshape(2, NUM_HIDDENS)
    flag = jnp.full((LANES,), first_sentence, dtype=jnp.int32)
    out = _sc_add(xf, seg2, flag)
    return out.reshape(X.shape)


# speedup vs baseline: 1.1685x; 1.1685x over previous
import functools

import jax
import jax.numpy as jnp
from jax import lax
from jax.experimental import pallas as pl
from jax.experimental.pallas import tpu as pltpu
from jax.experimental.pallas import tpu_sc as plsc

NUM_HIDDENS = 768
LANES = 16
SEG_SLICES = NUM_HIDDENS // LANES
NC, NS = 2, 16
NW = NC * NS
ROWS = 4 * 8192
ROWS_PER_W = ROWS // NW
CHUNK = 32
NPIECE = 2
PIECE = CHUNK // NPIECE
NBUF = 4
NCHUNKS = ROWS_PER_W // CHUNK


def _sc_add(xf, seg2, flag):
    mesh = plsc.VectorSubcoreMesh(core_axis_name="c", subcore_axis_name="s")

    @functools.partial(
        pl.kernel,
        mesh=mesh,
        out_type=jax.ShapeDtypeStruct((ROWS, NUM_HIDDENS), jnp.float32),
        scratch_types=[
            pltpu.VMEM((2, NUM_HIDDENS), jnp.float32),
            pltpu.VMEM((LANES,), jnp.int32),
        ] + [pltpu.VMEM((CHUNK, NUM_HIDDENS), jnp.float32)] * NBUF
          + [pltpu.SemaphoreType.DMA] * (2 * NBUF),
    )
    def k(x_hbm, seg_hbm, flag_hbm, out_hbm, seg_v, flag_v, *ring):
        bufs = ring[:NBUF]
        in_sems = ring[NBUF:2 * NBUF]
        out_sems = ring[2 * NBUF:]
        wid = lax.axis_index("s") * NC + lax.axis_index("c")
        pltpu.sync_copy(seg_hbm, seg_v)
        pltpu.sync_copy(flag_hbm, flag_v)
        f = flag_v[...] != 0
        segs = [
            jnp.where(f, seg_v[0, pl.ds(j * LANES, LANES)],
                      seg_v[1, pl.ds(j * LANES, LANES)])
            for j in range(SEG_SLICES)
        ]
        row0 = wid * ROWS_PER_W

        def in_copy(g, b):
            return pltpu.make_async_copy(
                x_hbm.at[pl.ds(row0 + g * CHUNK, CHUNK)], bufs[b], in_sems[b])

        def out_piece(g, b, h):
            return pltpu.make_async_copy(
                bufs[b].at[pl.ds(h * PIECE, PIECE)],
                out_hbm.at[pl.ds(row0 + g * CHUNK + h * PIECE, PIECE)],
                out_sems[b])

        def compute_piece(b, h):
            buf = bufs[b]

            def row_body(r, c):
                for j in range(SEG_SLICES):
                    sl = pl.ds(j * LANES, LANES)
                    plsc.addupdate(buf.at[r, sl], segs[j])
                return c

            lax.fori_loop(h * PIECE, (h + 1) * PIECE, row_body, 0)

        def process(g, b):
            in_copy(g, b).wait()
            for h in range(NPIECE):
                compute_piece(b, h)
                out_piece(g, b, h).start()

        def wait_out(g, b):
            for h in range(NPIECE):
                out_piece(g, b, h).wait()

        for g in range(NBUF - 1):
            in_copy(g, g).start()
        for g in range(NBUF):
            process(g, g)
            if g >= 1:
                wait_out(g - 1, g - 1)
            in_copy(g + NBUF - 1, (g + NBUF - 1) % NBUF).start()

        def group_body(p, c):
            for b in range(NBUF):
                g = p * NBUF + b
                process(g, b)
                wait_out(g - 1, (b - 1) % NBUF)
                in_copy(g + NBUF - 1, (b - 1) % NBUF).start()
            return c

        lax.fori_loop(1, NCHUNKS // NBUF - 1, group_body, 0)

        for g in range(NCHUNKS - NBUF, NCHUNKS):
            b = g % NBUF
            process(g, b)
            wait_out(g - 1, (b - 1) % NBUF)
            if g + NBUF - 1 < NCHUNKS:
                in_copy(g + NBUF - 1, (b - 1) % NBUF).start()
        wait_out(NCHUNKS - 1, (NCHUNKS - 1) % NBUF)

    return k(xf, seg2, flag)


def kernel(X, seg_emb, first_sentence):
    xf = X.reshape(ROWS, NUM_HIDDENS)
    seg2 = seg_emb.reshape(2, NUM_HIDDENS)
    flag = jnp.full((LANES,), first_sentence, dtype=jnp.int32)
    out = _sc_add(xf, seg2, flag)
    return out.reshape(X.shape)


# speedup vs baseline: 1.1777x; 1.0079x over previous
import functools

import jax
import jax.numpy as jnp
from jax import lax
from jax.experimental import pallas as pl
from jax.experimental.pallas import tpu as pltpu
from jax.experimental.pallas import tpu_sc as plsc

NUM_HIDDENS = 768
LANES = 16
SEG_SLICES = NUM_HIDDENS // LANES
NC, NS = 2, 16
NW = NC * NS
ROWS = 4 * 8192
ROWS_PER_W = ROWS // NW
CHUNK = 32
PIECES = ((0, 8), (8, 24))
NBUF = 4
NCHUNKS = ROWS_PER_W // CHUNK


def _sc_add(xf, seg2, flag):
    mesh = plsc.VectorSubcoreMesh(core_axis_name="c", subcore_axis_name="s")

    @functools.partial(
        pl.kernel,
        mesh=mesh,
        out_type=jax.ShapeDtypeStruct((ROWS, NUM_HIDDENS), jnp.float32),
        scratch_types=[
            pltpu.VMEM((2, NUM_HIDDENS), jnp.float32),
            pltpu.VMEM((LANES,), jnp.int32),
        ] + [pltpu.VMEM((CHUNK, NUM_HIDDENS), jnp.float32)] * NBUF
          + [pltpu.SemaphoreType.DMA] * (2 * NBUF),
    )
    def k(x_hbm, seg_hbm, flag_hbm, out_hbm, seg_v, flag_v, *ring):
        bufs = ring[:NBUF]
        in_sems = ring[NBUF:2 * NBUF]
        out_sems = ring[2 * NBUF:]
        wid = lax.axis_index("s") * NC + lax.axis_index("c")
        pltpu.sync_copy(seg_hbm, seg_v)
        pltpu.sync_copy(flag_hbm, flag_v)
        f = flag_v[...] != 0
        segs = [
            jnp.where(f, seg_v[0, pl.ds(j * LANES, LANES)],
                      seg_v[1, pl.ds(j * LANES, LANES)])
            for j in range(SEG_SLICES)
        ]
        row0 = wid * ROWS_PER_W

        def in_copy(g, b):
            return pltpu.make_async_copy(
                x_hbm.at[pl.ds(row0 + g * CHUNK, CHUNK)], bufs[b], in_sems[b])

        def out_piece(g, b, h):
            off, n = PIECES[h]
            return pltpu.make_async_copy(
                bufs[b].at[pl.ds(off, n)],
                out_hbm.at[pl.ds(row0 + g * CHUNK + off, n)],
                out_sems[b])

        def compute_piece(b, h):
            buf = bufs[b]
            off, n = PIECES[h]

            def row_body(r, c):
                for j in range(SEG_SLICES):
                    sl = pl.ds(j * LANES, LANES)
                    plsc.addupdate(buf.at[r, sl], segs[j])
                return c

            lax.fori_loop(off, off + n, row_body, 0)

        def process(g, b):
            in_copy(g, b).wait()
            for h in range(len(PIECES)):
                compute_piece(b, h)
                out_piece(g, b, h).start()

        def wait_out(g, b):
            for h in range(len(PIECES)):
                out_piece(g, b, h).wait()

        for g in range(NBUF - 1):
            in_copy(g, g).start()
        for g in range(NBUF):
            process(g, g)
            if g >= 1:
                wait_out(g - 1, g - 1)
            in_copy(g + NBUF - 1, (g + NBUF - 1) % NBUF).start()

        def group_body(p, c):
            for b in range(NBUF):
                g = p * NBUF + b
                process(g, b)
                wait_out(g - 1, (b - 1) % NBUF)
                in_copy(g + NBUF - 1, (b - 1) % NBUF).start()
            return c

        lax.fori_loop(1, NCHUNKS // NBUF - 1, group_body, 0)

        for g in range(NCHUNKS - NBUF, NCHUNKS):
            b = g % NBUF
            process(g, b)
            wait_out(g - 1, (b - 1) % NBUF)
            if g + NBUF - 1 < NCHUNKS:
                in_copy(g + NBUF - 1, (b - 1) % NBUF).start()
        wait_out(NCHUNKS - 1, (NCHUNKS - 1) % NBUF)

    return k(xf, seg2, flag)


def kernel(X, seg_emb, first_sentence):
    xf = X.reshape(ROWS, NUM_HIDDENS)
    seg2 = seg_emb.reshape(2, NUM_HIDDENS)
    flag = jnp.full((LANES,), first_sentence, dtype=jnp.int32)
    out = _sc_add(xf, seg2, flag)
    return out.reshape(X.shape)
